# baseline (speedup 1.0000x reference)
import functools

import jax
import jax.numpy as jnp
from jax import lax
from jax.experimental import pallas as pl
from jax.experimental.pallas import tpu as pltpu
from jax.experimental.pallas import tpu_sc as plsc

N = 1024
D = 64
E = 200000

_NW = 32
_CHUNK = 6272
_SEG = _CHUNK // 4


def _tc_body(z_ref, ei_hbm, w0_ref, b0_ref, w1_ref, b1_ref, bias_ref, s_hbm,
             idx_hbm, ei_v, idx_v, sb0, sb1, ei_sem, s_sem, idx_sem):
    ei_cp = pltpu.make_async_copy(ei_hbm, ei_v, ei_sem)
    ei_cp.start()
    z = z_ref[...]
    zbar = jnp.mean(z, axis=0, keepdims=True)
    c0 = jax.nn.relu(
        jnp.dot(zbar, w0_ref[...], preferred_element_type=jnp.float32)
        + b0_ref[...])
    c1 = (jnp.dot(zbar + c0, w1_ref[...], preferred_element_type=jnp.float32)
          + b1_ref[...])
    h = z + c0 + c1
    bufs = (sb0, sb1)
    copies = [None, None]
    for k in range(8):
        b = k % 2
        if copies[b] is not None:
            copies[b].wait()
        hk = h[128 * k:128 * (k + 1), :]
        gk = lax.dot_general(h, hk, (((1,), (1,)), ((), ())),
                             preferred_element_type=jnp.float32)
        bufs[b][...] = jax.nn.sigmoid(gk + bias_ref[0, 0])
        cp = pltpu.make_async_copy(bufs[b], s_hbm.at[pl.ds(1024 * k, 1024), :],
                                   s_sem.at[b])
        cp.start()
        copies[b] = cp
    ei_cp.wait()
    i = ei_v[0]
    j = ei_v[1]
    idx_v[...] = ((j >> 7) << 17) + (i << 7) + (j & 127)
    idx_cp = pltpu.make_async_copy(idx_v, idx_hbm, idx_sem)
    idx_cp.start()
    copies[0].wait()
    copies[1].wait()
    idx_cp.wait()


def _sc_gather(s_hbm, idx_hbm, out_hbm, idx_v, val_v, sem):
    nc = 2
    wid = lax.axis_index("s") * nc + lax.axis_index("c")
    base = jnp.minimum(wid * _CHUNK, E - _CHUNK)
    pltpu.sync_copy(idx_hbm.at[pl.ds(base, _CHUNK)], idx_v)
    pltpu.async_copy(s_hbm.at[idx_v], val_v, sem).wait()
    pltpu.sync_copy(val_v, out_hbm.at[pl.ds(base, _CHUNK)])


def kernel(z, edge_index, W0, b0, W1, b1, bias):
    s, idx = pl.pallas_call(
        _tc_body,
        out_shape=[
            jax.ShapeDtypeStruct((8 * N, 128), jnp.float32),
            jax.ShapeDtypeStruct((E,), jnp.int32),
        ],
        in_specs=[
            pl.BlockSpec(memory_space=pltpu.MemorySpace.VMEM),
            pl.BlockSpec(memory_space=pl.ANY),
            pl.BlockSpec(memory_space=pltpu.MemorySpace.VMEM),
            pl.BlockSpec(memory_space=pltpu.MemorySpace.VMEM),
            pl.BlockSpec(memory_space=pltpu.MemorySpace.VMEM),
            pl.BlockSpec(memory_space=pltpu.MemorySpace.VMEM),
            pl.BlockSpec(memory_space=pltpu.MemorySpace.VMEM),
        ],
        out_specs=[
            pl.BlockSpec(memory_space=pl.ANY),
            pl.BlockSpec(memory_space=pl.ANY),
        ],
        scratch_shapes=[
            pltpu.VMEM((2, E), jnp.int32),
            pltpu.VMEM((E,), jnp.int32),
            pltpu.VMEM((N, 128), jnp.float32),
            pltpu.VMEM((N, 128), jnp.float32),
            pltpu.SemaphoreType.DMA,
            pltpu.SemaphoreType.DMA((2,)),
            pltpu.SemaphoreType.DMA,
        ],
    )(z, edge_index, W0, b0.reshape(1, D), W1, b1.reshape(1, D),
      bias.reshape(1, 1))
    s = s.reshape(N * N)

    mesh = plsc.VectorSubcoreMesh(core_axis_name="c", subcore_axis_name="s")
    gather = functools.partial(
        pl.kernel,
        mesh=mesh,
        out_type=jax.ShapeDtypeStruct((E,), jnp.float32),
        scratch_types=[
            pltpu.VMEM((_CHUNK,), jnp.int32),
            pltpu.VMEM((_CHUNK,), jnp.float32),
            pltpu.SemaphoreType.DMA,
        ],
    )(_sc_gather)

    return gather(s, idx)


# speedup vs baseline: 1.0517x; 1.0517x over previous
import functools

import jax
import jax.numpy as jnp
from jax import lax
from jax.experimental import pallas as pl
from jax.experimental.pallas import tpu as pltpu
from jax.experimental.pallas import tpu_sc as plsc

N = 1024
D = 64
E = 200000

_NW = 32
_CHUNK = 6272
_SEG = _CHUNK // 4


def _tc_body(z_ref, ei_hbm, w0_ref, b0_ref, w1_ref, b1_ref, bias_ref, s_ref,
             idx_ref, src_v, dst_v, ei_sem):
    src_cp = pltpu.make_async_copy(ei_hbm.at[0], src_v, ei_sem.at[0])
    dst_cp = pltpu.make_async_copy(ei_hbm.at[1], dst_v, ei_sem.at[1])
    src_cp.start()
    dst_cp.start()
    z = z_ref[...]
    zbar = jnp.mean(z, axis=0, keepdims=True)
    c0 = jax.nn.relu(
        jnp.dot(zbar, w0_ref[...], preferred_element_type=jnp.float32)
        + b0_ref[...])
    c1 = (jnp.dot(zbar + c0, w1_ref[...], preferred_element_type=jnp.float32)
          + b1_ref[...])
    h = z + c0 + c1
    for k in range(8):
        hk = h[128 * k:128 * (k + 1), :]
        gk = lax.dot_general(h, hk, (((1,), (1,)), ((), ())),
                             preferred_element_type=jnp.float32)
        s_ref[1024 * k:1024 * (k + 1), :] = jax.nn.sigmoid(gk + bias_ref[0, 0])
    src_cp.wait()
    dst_cp.wait()
    i = src_v[...]
    j = dst_v[...]
    idx_ref[...] = ((j >> 7) << 17) + (i << 7) + (j & 127)


def _sc_gather(s_hbm, idx_hbm, out_hbm, idx_v, val_v, sem):
    nc = 2
    wid = lax.axis_index("s") * nc + lax.axis_index("c")
    base = jnp.minimum(wid * _CHUNK, E - _CHUNK)
    pltpu.sync_copy(idx_hbm.at[pl.ds(base, _CHUNK)], idx_v)
    pltpu.async_copy(s_hbm.at[idx_v], val_v, sem).wait()
    pltpu.sync_copy(val_v, out_hbm.at[pl.ds(base, _CHUNK)])


def kernel(z, edge_index, W0, b0, W1, b1, bias):
    s, idx = pl.pallas_call(
        _tc_body,
        out_shape=[
            jax.ShapeDtypeStruct((8 * N, 128), jnp.float32),
            jax.ShapeDtypeStruct((E,), jnp.int32),
        ],
        in_specs=[
            pl.BlockSpec(memory_space=pltpu.MemorySpace.VMEM),
            pl.BlockSpec(memory_space=pl.ANY),
            pl.BlockSpec(memory_space=pltpu.MemorySpace.VMEM),
            pl.BlockSpec(memory_space=pltpu.MemorySpace.VMEM),
            pl.BlockSpec(memory_space=pltpu.MemorySpace.VMEM),
            pl.BlockSpec(memory_space=pltpu.MemorySpace.VMEM),
            pl.BlockSpec(memory_space=pltpu.MemorySpace.VMEM),
        ],
        scratch_shapes=[
            pltpu.VMEM((E,), jnp.int32),
            pltpu.VMEM((E,), jnp.int32),
            pltpu.SemaphoreType.DMA((2,)),
        ],
    )(z, edge_index, W0, b0.reshape(1, D), W1, b1.reshape(1, D),
      bias.reshape(1, 1))
    s = s.reshape(N * N)

    mesh = plsc.VectorSubcoreMesh(core_axis_name="c", subcore_axis_name="s")
    gather = functools.partial(
        pl.kernel,
        mesh=mesh,
        out_type=jax.ShapeDtypeStruct((E,), jnp.float32),
        scratch_types=[
            pltpu.VMEM((_CHUNK,), jnp.int32),
            pltpu.VMEM((_CHUNK,), jnp.float32),
            pltpu.SemaphoreType.DMA,
        ],
    )(_sc_gather)

    return gather(s, idx)


# speedup vs baseline: 1.1185x; 1.0635x over previous
import functools

import jax
import jax.numpy as jnp
from jax import lax
from jax.experimental import pallas as pl
from jax.experimental.pallas import tpu as pltpu
from jax.experimental.pallas import tpu_sc as plsc

N = 1024
D = 64
E = 200000

_NW = 32
_CHUNK = 6272
_SEG = _CHUNK // 4


def _tc_body(zt_ref, ei_hbm, w0_ref, b0_ref, w1_ref, b1_ref, bias_ref, s_ref,
             idx_ref, src_v, dst_v, ei_sem):
    src_cp = pltpu.make_async_copy(ei_hbm.at[0], src_v, ei_sem.at[0])
    dst_cp = pltpu.make_async_copy(ei_hbm.at[1], dst_v, ei_sem.at[1])
    src_cp.start()
    dst_cp.start()
    zt = zt_ref[...]
    zbar = jnp.mean(zt, axis=1, keepdims=True).T
    c0 = jax.nn.relu(
        jnp.dot(zbar, w0_ref[...], preferred_element_type=jnp.float32)
        + b0_ref[...])
    c1 = (jnp.dot(zbar + c0, w1_ref[...], preferred_element_type=jnp.float32)
          + b1_ref[...])
    ht = zt + (c0 + c1).T
    for k in range(8):
        hk = ht[:, 128 * k:128 * (k + 1)]
        gk = lax.dot_general(ht, hk, (((0,), (0,)), ((), ())),
                             preferred_element_type=jnp.float32)
        s_ref[1024 * k:1024 * (k + 1), :] = jax.nn.sigmoid(gk + bias_ref[0, 0])
    src_cp.wait()
    dst_cp.wait()
    i = src_v[...]
    j = dst_v[...]
    idx_ref[...] = ((j >> 7) << 17) + (i << 7) + (j & 127)


def _sc_gather(s_hbm, idx_hbm, out_hbm, idx_v, val_v, sem):
    nc = 2
    wid = lax.axis_index("s") * nc + lax.axis_index("c")
    base = jnp.minimum(wid * _CHUNK, E - _CHUNK)
    pltpu.sync_copy(idx_hbm.at[pl.ds(base, _CHUNK)], idx_v)
    pltpu.async_copy(s_hbm.at[idx_v], val_v, sem).wait()
    pltpu.sync_copy(val_v, out_hbm.at[pl.ds(base, _CHUNK)])


def kernel(z, edge_index, W0, b0, W1, b1, bias):
    s, idx = pl.pallas_call(
        _tc_body,
        out_shape=[
            jax.ShapeDtypeStruct((8 * N, 128), jnp.float32),
            jax.ShapeDtypeStruct((E,), jnp.int32),
        ],
        in_specs=[
            pl.BlockSpec(memory_space=pltpu.MemorySpace.VMEM),
            pl.BlockSpec(memory_space=pl.ANY),
            pl.BlockSpec(memory_space=pltpu.MemorySpace.VMEM),
            pl.BlockSpec(memory_space=pltpu.MemorySpace.VMEM),
            pl.BlockSpec(memory_space=pltpu.MemorySpace.VMEM),
            pl.BlockSpec(memory_space=pltpu.MemorySpace.VMEM),
            pl.BlockSpec(memory_space=pltpu.MemorySpace.VMEM),
        ],
        scratch_shapes=[
            pltpu.VMEM((E,), jnp.int32),
            pltpu.VMEM((E,), jnp.int32),
            pltpu.SemaphoreType.DMA((2,)),
        ],
    )(z.T, edge_index, W0, b0.reshape(1, D), W1, b1.reshape(1, D),
      bias.reshape(1, 1))
    s = s.reshape(N * N)

    mesh = plsc.VectorSubcoreMesh(core_axis_name="c", subcore_axis_name="s")
    gather = functools.partial(
        pl.kernel,
        mesh=mesh,
        out_type=jax.ShapeDtypeStruct((E,), jnp.float32),
        scratch_types=[
            pltpu.VMEM((_CHUNK,), jnp.int32),
            pltpu.VMEM((_CHUNK,), jnp.float32),
            pltpu.SemaphoreType.DMA,
        ],
    )(_sc_gather)

    return gather(s, idx)


# speedup vs baseline: 1.1589x; 1.0361x over previous
import functools

import jax
import jax.numpy as jnp
from jax import lax
from jax.experimental import pallas as pl
from jax.experimental.pallas import tpu as pltpu
from jax.experimental.pallas import tpu_sc as plsc

N = 1024
D = 64
E = 200000

_NW = 32
_CHUNK = 6272
_SEG = _CHUNK // 4


def _tc_body(zt_ref, ei_hbm, w0_ref, b0_ref, w1_ref, b1_ref, bias_ref, s_ref,
             idx_ref, src_v, dst_v, ei_sem):
    src_cp = pltpu.make_async_copy(ei_hbm.at[0], src_v, ei_sem.at[0])
    dst_cp = pltpu.make_async_copy(ei_hbm.at[1], dst_v, ei_sem.at[1])
    src_cp.start()
    dst_cp.start()
    zt = zt_ref[...]
    zbar = jnp.mean(zt, axis=1, keepdims=True).T
    c0 = jax.nn.relu(
        jnp.dot(zbar, w0_ref[...], preferred_element_type=jnp.float32)
        + b0_ref[...])
    c1 = (jnp.dot(zbar + c0, w1_ref[...], preferred_element_type=jnp.float32)
          + b1_ref[...])
    ht = zt + (c0 + c1).T
    for k in range(8):
        hk = ht[:, 128 * k:128 * (k + 1)]
        gk = lax.dot_general(ht, hk, (((0,), (0,)), ((), ())),
                             preferred_element_type=jnp.float32)
        s_ref[1024 * k:1024 * (k + 1), :] = jax.nn.sigmoid(gk + bias_ref[0, 0])
    src_cp.wait()
    dst_cp.wait()
    i = src_v[...]
    j = dst_v[...]
    idx_ref[...] = ((j >> 7) << 17) + (i << 7) + (j & 127)


def _sc_gather(s_hbm, idx_hbm, out_hbm, idx_v, val_v, s_sh, sem, sem2):
    nc = 2
    sid = lax.axis_index("s")
    wid = sid * nc + lax.axis_index("c")
    base = jnp.minimum(wid * _CHUNK, E - _CHUNK)
    slab = N * N // 16
    stage = pltpu.make_async_copy(s_hbm.at[pl.ds(sid * slab, slab)],
                                  s_sh.at[pl.ds(sid * slab, slab)], sem2)
    stage.start()
    pltpu.sync_copy(idx_hbm.at[pl.ds(base, _CHUNK)], idx_v)
    stage.wait()
    plsc.subcore_barrier()
    pltpu.async_copy(s_sh.at[idx_v], val_v, sem).wait()
    pltpu.sync_copy(val_v, out_hbm.at[pl.ds(base, _CHUNK)])


def kernel(z, edge_index, W0, b0, W1, b1, bias):
    s, idx = pl.pallas_call(
        _tc_body,
        out_shape=[
            jax.ShapeDtypeStruct((8 * N, 128), jnp.float32),
            jax.ShapeDtypeStruct((E,), jnp.int32),
        ],
        in_specs=[
            pl.BlockSpec(memory_space=pltpu.MemorySpace.VMEM),
            pl.BlockSpec(memory_space=pl.ANY),
            pl.BlockSpec(memory_space=pltpu.MemorySpace.VMEM),
            pl.BlockSpec(memory_space=pltpu.MemorySpace.VMEM),
            pl.BlockSpec(memory_space=pltpu.MemorySpace.VMEM),
            pl.BlockSpec(memory_space=pltpu.MemorySpace.VMEM),
            pl.BlockSpec(memory_space=pltpu.MemorySpace.VMEM),
        ],
        scratch_shapes=[
            pltpu.VMEM((E,), jnp.int32),
            pltpu.VMEM((E,), jnp.int32),
            pltpu.SemaphoreType.DMA((2,)),
        ],
    )(z.T, edge_index, W0, b0.reshape(1, D), W1, b1.reshape(1, D),
      bias.reshape(1, 1))
    s = s.reshape(N * N)

    mesh = plsc.VectorSubcoreMesh(core_axis_name="c", subcore_axis_name="s")
    gather = functools.partial(
        pl.kernel,
        mesh=mesh,
        out_type=jax.ShapeDtypeStruct((E,), jnp.float32),
        scratch_types=[
            pltpu.VMEM((_CHUNK,), jnp.int32),
            pltpu.VMEM((_CHUNK,), jnp.float32),
            pltpu.VMEM_SHARED((N * N,), jnp.float32),
            pltpu.SemaphoreType.DMA,
            pltpu.SemaphoreType.DMA,
        ],
    )(_sc_gather)

    return gather(s, idx)


# speedup vs baseline: 1.1601x; 1.0010x over previous
import functools

import jax
import jax.numpy as jnp
from jax import lax
from jax.experimental import pallas as pl
from jax.experimental.pallas import tpu as pltpu
from jax.experimental.pallas import tpu_sc as plsc

N = 1024
D = 64
E = 200000

_NW = 32
_CHUNK = 6272
_SEG = _CHUNK // 4


def _tc_body(zt_ref, ei_hbm, w0_ref, b0_ref, w1_ref, b1_ref, bias_ref, s_ref,
             idx_ref, src_v, dst_v, ei_sem):
    src_cp = pltpu.make_async_copy(ei_hbm.at[0], src_v, ei_sem.at[0])
    dst_cp = pltpu.make_async_copy(ei_hbm.at[1], dst_v, ei_sem.at[1])
    src_cp.start()
    dst_cp.start()
    zt = zt_ref[...]
    zbar = jnp.mean(zt, axis=1, keepdims=True).T
    c0 = jax.nn.relu(
        jnp.dot(zbar, w0_ref[...], preferred_element_type=jnp.float32)
        + b0_ref[...])
    c1 = (jnp.dot(zbar + c0, w1_ref[...], preferred_element_type=jnp.float32)
          + b1_ref[...])
    ht = zt + (c0 + c1).T
    for k in range(8):
        hk = ht[:, 128 * k:128 * (k + 1)]
        gk = lax.dot_general(ht, hk, (((0,), (0,)), ((), ())),
                             preferred_element_type=jnp.float32)
        s_ref[1024 * k:1024 * (k + 1), :] = jax.nn.sigmoid(gk + bias_ref[0, 0])
    src_cp.wait()
    dst_cp.wait()
    i = src_v[...]
    j = dst_v[...]
    idx_ref[...] = ((j >> 7) << 17) + (i << 7) + (j & 127)


def _sc_gather(s_hbm, idx_hbm, out_hbm, idx_v, val_v, s_sh, sem, sem2):
    nc = 2
    sid = lax.axis_index("s")
    wid = sid * nc + lax.axis_index("c")
    base = jnp.minimum(wid * _CHUNK, E - _CHUNK)
    slab = N * N // 16
    stage = pltpu.make_async_copy(s_hbm.at[pl.ds(sid * slab, slab)],
                                  s_sh.at[pl.ds(sid * slab, slab)], sem2)
    stage.start()
    pltpu.sync_copy(idx_hbm.at[pl.ds(base, _CHUNK)], idx_v)
    stage.wait()
    plsc.subcore_barrier()
    half = _CHUNK // 2
    g0 = pltpu.async_copy(s_sh.at[idx_v.at[pl.ds(0, half)]],
                          val_v.at[pl.ds(0, half)], sem)
    g1 = pltpu.async_copy(s_sh.at[idx_v.at[pl.ds(half, half)]],
                          val_v.at[pl.ds(half, half)], sem)
    g0.wait()
    pltpu.sync_copy(val_v.at[pl.ds(0, half)], out_hbm.at[pl.ds(base, half)])
    g1.wait()
    pltpu.sync_copy(val_v.at[pl.ds(half, half)],
                    out_hbm.at[pl.ds(base + half, half)])


def kernel(z, edge_index, W0, b0, W1, b1, bias):
    s, idx = pl.pallas_call(
        _tc_body,
        out_shape=[
            jax.ShapeDtypeStruct((8 * N, 128), jnp.float32),
            jax.ShapeDtypeStruct((E,), jnp.int32),
        ],
        in_specs=[
            pl.BlockSpec(memory_space=pltpu.MemorySpace.VMEM),
            pl.BlockSpec(memory_space=pl.ANY),
            pl.BlockSpec(memory_space=pltpu.MemorySpace.VMEM),
            pl.BlockSpec(memory_space=pltpu.MemorySpace.VMEM),
            pl.BlockSpec(memory_space=pltpu.MemorySpace.VMEM),
            pl.BlockSpec(memory_space=pltpu.MemorySpace.VMEM),
            pl.BlockSpec(memory_space=pltpu.MemorySpace.VMEM),
        ],
        scratch_shapes=[
            pltpu.VMEM((E,), jnp.int32),
            pltpu.VMEM((E,), jnp.int32),
            pltpu.SemaphoreType.DMA((2,)),
        ],
    )(z.T, edge_index, W0, b0.reshape(1, D), W1, b1.reshape(1, D),
      bias.reshape(1, 1))
    s = s.reshape(N * N)

    mesh = plsc.VectorSubcoreMesh(core_axis_name="c", subcore_axis_name="s")
    gather = functools.partial(
        pl.kernel,
        mesh=mesh,
        out_type=jax.ShapeDtypeStruct((E,), jnp.float32),
        scratch_types=[
            pltpu.VMEM((_CHUNK,), jnp.int32),
            pltpu.VMEM((_CHUNK,), jnp.float32),
            pltpu.VMEM_SHARED((N * N,), jnp.float32),
            pltpu.SemaphoreType.DMA,
            pltpu.SemaphoreType.DMA,
        ],
    )(_sc_gather)

    return gather(s, idx)
